# Initial kernel scaffold; baseline (speedup 1.0000x reference)
import jax
import jax.numpy as jnp
from jax.experimental import pallas as pl


def kernel(x, edge_index, W1, b1, W2, b2, Wlin, blin):
    raise NotImplementedError("write your pallas kernel here")


# speedup vs baseline: 32.6373x; 32.6373x over previous
import jax
import jax.numpy as jnp
from jax import lax
from jax.experimental import pallas as pl
from jax.experimental.pallas import tpu as pltpu
from jax.experimental.pallas import tpu_sc as plsc

N = 10000
D = 128
E = 320000
NC = 2
NS = 16
BA = 80
NB = (E // (NC * NS)) // BA
NBF = (E // NS) // BA
ET = E // NS

f32 = jnp.float32
i32 = jnp.int32

_MESH = plsc.VectorSubcoreMesh(core_axis_name="c", subcore_axis_name="s")


def _zero_1d(ref, nvec):
    def z(i, _):
        ref[pl.ds(i * 16, 16)] = jnp.zeros((16,), f32)
        return 0
    lax.fori_loop(0, nvec, z, 0)


def _deg_body(dst_hbm, deg_out, dst_v, ones_v, buf_v, acc_s):
    cid = lax.axis_index("c")
    sid = lax.axis_index("s")
    _zero_1d(buf_v, 63)
    for i in range(BA // 16):
        ones_v[pl.ds(i * 16, 16)] = jnp.ones((16,), f32)

    @pl.when(sid < 10)
    def _zero_acc():
        pltpu.sync_copy(buf_v.at[pl.ds(0, 1000)], acc_s.at[pl.ds(sid * 1000, 1000)])

    plsc.subcore_barrier()
    pltpu.sync_copy(dst_hbm.at[cid, sid], dst_v)

    def body(j, _):
        pltpu.sync_copy(ones_v, acc_s.at[dst_v.at[j]], add=True)
        return 0
    lax.fori_loop(0, NB, body, 0)
    plsc.subcore_barrier()

    @pl.when(sid < 10)
    def _drain():
        pltpu.sync_copy(acc_s.at[pl.ds(sid * 1000, 1000)], buf_v.at[pl.ds(0, 1000)])
        pltpu.sync_copy(buf_v.at[pl.ds(0, 1000)],
                        deg_out.at[pl.ds(cid * N + sid * 1000, 1000)])


_deg = pl.kernel(
    _deg_body,
    out_type=jax.ShapeDtypeStruct((2 * N,), f32),
    mesh=_MESH,
    scratch_types=[
        pltpu.VMEM((NB, BA), i32),
        pltpu.VMEM((BA,), f32),
        pltpu.VMEM((1008,), f32),
        pltpu.VMEM_SHARED((N,), f32),
    ],
)


R_BLK = 2000


def _prescale_body(x_ref, d0_ref, d1_ref, xt_ref, dinv_ref):
    dv = lax.rsqrt(d0_ref[...] + d1_ref[...] + 1.0)
    dinv_ref[...] = dv
    xt_ref[...] = x_ref[...] * dv


_prescale = pl.pallas_call(
    _prescale_body,
    grid=(N // R_BLK,),
    in_specs=[
        pl.BlockSpec((R_BLK, D), lambda i: (i, 0)),
        pl.BlockSpec((R_BLK, 1), lambda i: (i, 0)),
        pl.BlockSpec((R_BLK, 1), lambda i: (i, 0)),
    ],
    out_specs=[
        pl.BlockSpec((R_BLK, D), lambda i: (i, 0)),
        pl.BlockSpec((R_BLK, 1), lambda i: (i, 0)),
    ],
    out_shape=[
        jax.ShapeDtypeStruct((N, D), f32),
        jax.ShapeDtypeStruct((N, 1), f32),
    ],
)


def _prop_body(xt_hbm, src_hbm, dst_hbm, y_out,
               src_c, dst_c, rows0, rows1, acc_s, sem0, sem1):
    cid = lax.axis_index("c")
    sid = lax.axis_index("s")

    def zb(i, _):
        rows0[i // 8, pl.ds((i % 8) * 16, 16)] = jnp.zeros((16,), f32)
        return 0
    lax.fori_loop(0, BA * 8, zb, 0)

    @pl.when(sid < 10)
    def _zacc():
        def zacc(k, _):
            pltpu.sync_copy(rows0, acc_s.at[pl.ds(sid * 1000 + k * BA, BA)])
            return 0
        lax.fori_loop(0, 12, zacc, 0)
        pltpu.sync_copy(rows0.at[pl.ds(0, 40)], acc_s.at[pl.ds(sid * 1000 + 960, 40)])

    plsc.subcore_barrier()

    def chunk(m, _):
        pltpu.sync_copy(src_hbm.at[cid, sid, m], src_c)
        pltpu.sync_copy(dst_hbm.at[cid, sid, m], dst_c)

        def step(i, _):
            j = 2 * i
            d0 = pltpu.async_copy(xt_hbm.at[src_c.at[j]], rows0, sem0)
            d1 = pltpu.async_copy(xt_hbm.at[src_c.at[j + 1]], rows1, sem1)
            d0.wait()
            pltpu.sync_copy(rows0, acc_s.at[dst_c.at[j]], add=True)
            d1.wait()
            pltpu.sync_copy(rows1, acc_s.at[dst_c.at[j + 1]], add=True)
            return 0
        lax.fori_loop(0, 12, step, 0)
        pltpu.async_copy(xt_hbm.at[src_c.at[24]], rows0, sem0).wait()
        pltpu.sync_copy(rows0, acc_s.at[dst_c.at[24]], add=True)
        return 0
    lax.fori_loop(0, 5, chunk, 0)
    plsc.subcore_barrier()

    @pl.when(sid < 10)
    def _drain():
        def drain(k, _):
            base = sid * 1000 + k * BA
            pltpu.sync_copy(acc_s.at[pl.ds(base, BA)], rows0)
            pltpu.sync_copy(rows0, y_out.at[pl.ds(cid * N + base, BA)])
            return 0
        lax.fori_loop(0, 12, drain, 0)
        base = sid * 1000 + 960
        pltpu.sync_copy(acc_s.at[pl.ds(base, 40)], rows0.at[pl.ds(0, 40)])
        pltpu.sync_copy(rows0.at[pl.ds(0, 40)], y_out.at[pl.ds(cid * N + base, 40)])


_prop = pl.kernel(
    _prop_body,
    out_type=jax.ShapeDtypeStruct((2 * N, D), f32),
    mesh=_MESH,
    scratch_types=[
        pltpu.VMEM((25, BA), i32),
        pltpu.VMEM((25, BA), i32),
        pltpu.VMEM((BA, D), f32),
        pltpu.VMEM((BA, D), f32),
        pltpu.VMEM_SHARED((N, D), f32),
        pltpu.SemaphoreType.DMA,
        pltpu.SemaphoreType.DMA,
    ],
)


def _dense_body(y0_ref, y1_ref, xt_ref, dv_ref, W1_ref, b1_ref, W2_ref,
                Wl_ref, b2_ref, bl_ref, s_ref, q_ref):
    dv = dv_ref[...]
    y = dv * (y0_ref[...] + y1_ref[...] + xt_ref[...])
    h = jnp.maximum(
        jnp.dot(y, W1_ref[...], preferred_element_type=f32) + b1_ref[...], 0.0)
    w = jnp.dot(W2_ref[...], Wl_ref[...], preferred_element_type=f32)
    st = dv * jnp.dot(h, w, preferred_element_type=f32)
    s_ref[...] = st
    c = jnp.dot(b2_ref[...], Wl_ref[...], preferred_element_type=f32) + bl_ref[...]
    q_ref[...] = dv * st + c


_dense = pl.pallas_call(
    _dense_body,
    grid=(N // R_BLK,),
    in_specs=[
        pl.BlockSpec((R_BLK, D), lambda i: (i, 0)),
        pl.BlockSpec((R_BLK, D), lambda i: (i, 0)),
        pl.BlockSpec((R_BLK, D), lambda i: (i, 0)),
        pl.BlockSpec((R_BLK, 1), lambda i: (i, 0)),
        pl.BlockSpec((D, D), lambda i: (0, 0)),
        pl.BlockSpec((1, D), lambda i: (0, 0)),
        pl.BlockSpec((D, D), lambda i: (0, 0)),
        pl.BlockSpec((D, 1), lambda i: (0, 0)),
        pl.BlockSpec((1, D), lambda i: (0, 0)),
        pl.BlockSpec((1, 1), lambda i: (0, 0)),
    ],
    out_specs=[
        pl.BlockSpec((R_BLK, 1), lambda i: (i, 0)),
        pl.BlockSpec((R_BLK, 1), lambda i: (i, 0)),
    ],
    out_shape=[
        jax.ShapeDtypeStruct((N, 1), f32),
        jax.ShapeDtypeStruct((N, 1), f32),
    ],
)


def _final_body(st_hbm, dinv_hbm, q_hbm, src_hbm, dst_hbm, out_hbm,
                s_v, src_v, dst_v, vals_v, zz_v, zbuf, dvbuf, qbuf, obuf, acc_s):
    cid = lax.axis_index("c")
    sid = lax.axis_index("s")

    @pl.when(cid == 0)
    def _core0():
        _zero_1d(zz_v, 63)

        @pl.when(sid < 10)
        def _zero_acc():
            pltpu.sync_copy(zz_v.at[pl.ds(0, 1000)], acc_s.at[pl.ds(sid * 1000, 1000)])

        plsc.subcore_barrier()
        pltpu.sync_copy(st_hbm, s_v)
        pltpu.sync_copy(src_hbm.at[pl.ds(sid * ET, ET)], src_v)
        pltpu.sync_copy(dst_hbm.at[sid], dst_v)

        def g(i, _):
            idx = src_v[pl.ds(i * 16, 16)]
            vals_v[pl.ds(i * 16, 16)] = plsc.load_gather(s_v, [idx])
            return 0
        lax.fori_loop(0, ET // 16, g, 0)

        def sc(j, _):
            pltpu.sync_copy(vals_v.at[pl.ds(j * BA, BA)], acc_s.at[dst_v.at[j]], add=True)
            return 0
        lax.fori_loop(0, NBF, sc, 0)
        plsc.subcore_barrier()

        nbase = sid * 640

        @pl.when(sid < 15)
        def _ld_full():
            pltpu.sync_copy(acc_s.at[pl.ds(nbase, 640)], zbuf)
            pltpu.sync_copy(dinv_hbm.at[pl.ds(nbase, 640)], dvbuf)
            pltpu.sync_copy(q_hbm.at[pl.ds(nbase, 640)], qbuf)

        @pl.when(sid == 15)
        def _ld_tail():
            pltpu.sync_copy(acc_s.at[pl.ds(9600, 400)], zbuf.at[pl.ds(0, 400)])
            pltpu.sync_copy(dinv_hbm.at[pl.ds(9600, 400)], dvbuf.at[pl.ds(0, 400)])
            pltpu.sync_copy(q_hbm.at[pl.ds(9600, 400)], qbuf.at[pl.ds(0, 400)])

        nvec = jnp.where(sid < 15, 40, 25)

        def fin(i, _):
            t = dvbuf[pl.ds(i * 16, 16)] * zbuf[pl.ds(i * 16, 16)] + qbuf[pl.ds(i * 16, 16)]
            obuf[pl.ds(i * 16, 16)] = 1.0 / (1.0 + jnp.exp(-t))
            return 0
        lax.fori_loop(0, nvec, fin, 0)

        @pl.when(sid < 15)
        def _st_full():
            pltpu.sync_copy(obuf, out_hbm.at[pl.ds(nbase, 640)])

        @pl.when(sid == 15)
        def _st_tail():
            pltpu.sync_copy(obuf.at[pl.ds(0, 400)], out_hbm.at[pl.ds(9600, 400)])


_final = pl.kernel(
    _final_body,
    out_type=jax.ShapeDtypeStruct((N,), f32),
    mesh=_MESH,
    compiler_params=pltpu.CompilerParams(needs_layout_passes=False),
    scratch_types=[
        pltpu.VMEM((N,), f32),
        pltpu.VMEM((ET,), i32),
        pltpu.VMEM((NBF, BA), i32),
        pltpu.VMEM((ET,), f32),
        pltpu.VMEM((1008,), f32),
        pltpu.VMEM((640,), f32),
        pltpu.VMEM((640,), f32),
        pltpu.VMEM((640,), f32),
        pltpu.VMEM((640,), f32),
        pltpu.VMEM_SHARED((N,), f32),
    ],
)


def kernel(x, edge_index, W1, b1, W2, b2, Wlin, blin):
    src = edge_index[0].astype(i32)
    dst = edge_index[1].astype(i32)
    src3 = src.reshape(NC, NS, NB, BA)
    dst3 = dst.reshape(NC, NS, NB, BA)
    src5d = src.reshape(NC, NS, 5, 25, BA)
    dst5d = dst.reshape(NC, NS, 5, 25, BA)
    dst5 = dst.reshape(NS, NBF, BA)

    deg = _deg(dst3)
    d0 = deg[:N].reshape(N, 1)
    d1 = deg[N:].reshape(N, 1)
    xt, dinv = _prescale(x, d0, d1)
    yp = _prop(xt, src5d, dst5d)
    y0 = yp[:N]
    y1 = yp[N:]
    st, q = _dense(y0, y1, xt, dinv, W1, b1.reshape(1, D), W2,
                   Wlin, b2.reshape(1, D), blin.reshape(1, 1))
    out = _final(st.reshape(N), dinv.reshape(N), q.reshape(N), src, dst5)
    return out.reshape(N, 1)


# speedup vs baseline: 36.6319x; 1.1224x over previous
import jax
import jax.numpy as jnp
from jax import lax
from jax.experimental import pallas as pl
from jax.experimental.pallas import tpu as pltpu
from jax.experimental.pallas import tpu_sc as plsc

N = 10000
D = 128
E = 320000
NC = 2
NS = 16
BA = 80
NB = (E // (NC * NS)) // BA
NBF = (E // NS) // BA
ET = E // NS

f32 = jnp.float32
i32 = jnp.int32

_MESH = plsc.VectorSubcoreMesh(core_axis_name="c", subcore_axis_name="s")


def _zero_1d(ref, nvec):
    def z(i, _):
        ref[pl.ds(i * 16, 16)] = jnp.zeros((16,), f32)
        return 0
    lax.fori_loop(0, nvec, z, 0)


def _deg_body(dst_hbm, deg_out, dst_v, ones_v, buf_v, acc_s,
              semd0, semd1, semd2, semd3):
    cid = lax.axis_index("c")
    sid = lax.axis_index("s")
    _zero_1d(buf_v, 63)
    for i in range(BA // 16):
        ones_v[pl.ds(i * 16, 16)] = jnp.ones((16,), f32)

    @pl.when(sid < 10)
    def _zero_acc():
        pltpu.sync_copy(buf_v.at[pl.ds(0, 1000)], acc_s.at[pl.ds(sid * 1000, 1000)])

    plsc.subcore_barrier()
    pltpu.sync_copy(dst_hbm.at[cid, sid], dst_v)

    semd = [semd0, semd1, semd2, semd3]

    def body(i, _):
        j = 4 * i
        for p in range(4):
            @pl.when(i > 0)
            def _w(p=p):
                pltpu.make_async_copy(ones_v, acc_s.at[dst_v.at[j + p]], semd[p]).wait()
            pltpu.async_copy(ones_v, acc_s.at[dst_v.at[j + p]], semd[p], add=True)
        return 0
    lax.fori_loop(0, 31, body, 0)
    pltpu.make_async_copy(ones_v, acc_s.at[dst_v.at[124]], semd0).wait()
    pltpu.async_copy(ones_v, acc_s.at[dst_v.at[124]], semd0, add=True)
    for p in range(4):
        pltpu.make_async_copy(ones_v, acc_s.at[dst_v.at[124]], semd[p]).wait()
    plsc.subcore_barrier()

    @pl.when(sid < 10)
    def _drain():
        pltpu.sync_copy(acc_s.at[pl.ds(sid * 1000, 1000)], buf_v.at[pl.ds(0, 1000)])
        pltpu.sync_copy(buf_v.at[pl.ds(0, 1000)],
                        deg_out.at[pl.ds(cid * N + sid * 1000, 1000)])


_deg = pl.kernel(
    _deg_body,
    out_type=jax.ShapeDtypeStruct((2 * N,), f32),
    mesh=_MESH,
    scratch_types=[
        pltpu.VMEM((NB, BA), i32),
        pltpu.VMEM((BA,), f32),
        pltpu.VMEM((1008,), f32),
        pltpu.VMEM_SHARED((N,), f32),
        pltpu.SemaphoreType.DMA,
        pltpu.SemaphoreType.DMA,
        pltpu.SemaphoreType.DMA,
        pltpu.SemaphoreType.DMA,
    ],
)


R_BLK = 2000


def _prescale_body(x_ref, d0_ref, d1_ref, xt_ref, dinv_ref):
    dv = lax.rsqrt(d0_ref[...] + d1_ref[...] + 1.0)
    dinv_ref[...] = dv
    xt_ref[...] = x_ref[...] * dv


_prescale = pl.pallas_call(
    _prescale_body,
    grid=(N // R_BLK,),
    in_specs=[
        pl.BlockSpec((R_BLK, D), lambda i: (i, 0)),
        pl.BlockSpec((R_BLK, 1), lambda i: (i, 0)),
        pl.BlockSpec((R_BLK, 1), lambda i: (i, 0)),
    ],
    out_specs=[
        pl.BlockSpec((R_BLK, D), lambda i: (i, 0)),
        pl.BlockSpec((R_BLK, 1), lambda i: (i, 0)),
    ],
    out_shape=[
        jax.ShapeDtypeStruct((N, D), f32),
        jax.ShapeDtypeStruct((N, 1), f32),
    ],
)


def _prop_body(xt_hbm, src_hbm, dst_hbm, y_out,
               src_c, dst_c, rows0, rows1, acc_s, semg0, semg1, sems0, sems1):
    cid = lax.axis_index("c")
    sid = lax.axis_index("s")

    def zb(i, _):
        rows0[i // 8, pl.ds((i % 8) * 16, 16)] = jnp.zeros((16,), f32)
        return 0
    lax.fori_loop(0, BA * 8, zb, 0)

    @pl.when(sid < 10)
    def _zacc():
        def zacc(k, _):
            pltpu.sync_copy(rows0, acc_s.at[pl.ds(sid * 1000 + k * BA, BA)])
            return 0
        lax.fori_loop(0, 12, zacc, 0)
        pltpu.sync_copy(rows0.at[pl.ds(0, 40)], acc_s.at[pl.ds(sid * 1000 + 960, 40)])

    plsc.subcore_barrier()

    def chunk(m, _):
        pltpu.sync_copy(src_hbm.at[cid, sid, m], src_c)
        pltpu.sync_copy(dst_hbm.at[cid, sid, m], dst_c)
        pltpu.async_copy(xt_hbm.at[src_c.at[0]], rows0, semg0)

        def step(i, _):
            j = 2 * i
            pltpu.make_async_copy(xt_hbm.at[src_c.at[j]], rows0, semg0).wait()

            @pl.when(i > 0)
            def _ws1():
                pltpu.make_async_copy(rows1, acc_s.at[dst_c.at[j]], sems1).wait()

            pltpu.async_copy(xt_hbm.at[src_c.at[j + 1]], rows1, semg1)
            sd = pltpu.async_copy(rows0, acc_s.at[dst_c.at[j]], sems0, add=True)
            pltpu.make_async_copy(xt_hbm.at[src_c.at[j + 1]], rows1, semg1).wait()
            sd.wait()
            pltpu.async_copy(xt_hbm.at[src_c.at[j + 2]], rows0, semg0)
            pltpu.async_copy(rows1, acc_s.at[dst_c.at[j + 1]], sems1, add=True)
            return 0
        lax.fori_loop(0, 12, step, 0)
        pltpu.make_async_copy(xt_hbm.at[src_c.at[24]], rows0, semg0).wait()
        pltpu.make_async_copy(rows1, acc_s.at[dst_c.at[24]], sems1).wait()
        pltpu.async_copy(rows0, acc_s.at[dst_c.at[24]], sems0, add=True).wait()
        return 0
    lax.fori_loop(0, 5, chunk, 0)
    plsc.subcore_barrier()

    @pl.when(sid < 10)
    def _drain():
        def drain(k, _):
            base = sid * 1000 + k * BA
            pltpu.sync_copy(acc_s.at[pl.ds(base, BA)], rows0)
            pltpu.sync_copy(rows0, y_out.at[pl.ds(cid * N + base, BA)])
            return 0
        lax.fori_loop(0, 12, drain, 0)
        base = sid * 1000 + 960
        pltpu.sync_copy(acc_s.at[pl.ds(base, 40)], rows0.at[pl.ds(0, 40)])
        pltpu.sync_copy(rows0.at[pl.ds(0, 40)], y_out.at[pl.ds(cid * N + base, 40)])


_prop = pl.kernel(
    _prop_body,
    out_type=jax.ShapeDtypeStruct((2 * N, D), f32),
    mesh=_MESH,
    scratch_types=[
        pltpu.VMEM((25, BA), i32),
        pltpu.VMEM((25, BA), i32),
        pltpu.VMEM((BA, D), f32),
        pltpu.VMEM((BA, D), f32),
        pltpu.VMEM_SHARED((N, D), f32),
        pltpu.SemaphoreType.DMA,
        pltpu.SemaphoreType.DMA,
        pltpu.SemaphoreType.DMA,
        pltpu.SemaphoreType.DMA,
    ],
)


def _dense_body(y0_ref, y1_ref, xt_ref, dv_ref, W1_ref, b1_ref, W2_ref,
                Wl_ref, b2_ref, bl_ref, s_ref, q_ref):
    dv = dv_ref[...]
    y = dv * (y0_ref[...] + y1_ref[...] + xt_ref[...])
    h = jnp.maximum(
        jnp.dot(y, W1_ref[...], preferred_element_type=f32) + b1_ref[...], 0.0)
    w = jnp.dot(W2_ref[...], Wl_ref[...], preferred_element_type=f32)
    st = dv * jnp.dot(h, w, preferred_element_type=f32)
    s_ref[...] = st
    c = jnp.dot(b2_ref[...], Wl_ref[...], preferred_element_type=f32) + bl_ref[...]
    q_ref[...] = dv * st + c


_dense = pl.pallas_call(
    _dense_body,
    grid=(N // R_BLK,),
    in_specs=[
        pl.BlockSpec((R_BLK, D), lambda i: (i, 0)),
        pl.BlockSpec((R_BLK, D), lambda i: (i, 0)),
        pl.BlockSpec((R_BLK, D), lambda i: (i, 0)),
        pl.BlockSpec((R_BLK, 1), lambda i: (i, 0)),
        pl.BlockSpec((D, D), lambda i: (0, 0)),
        pl.BlockSpec((1, D), lambda i: (0, 0)),
        pl.BlockSpec((D, D), lambda i: (0, 0)),
        pl.BlockSpec((D, 1), lambda i: (0, 0)),
        pl.BlockSpec((1, D), lambda i: (0, 0)),
        pl.BlockSpec((1, 1), lambda i: (0, 0)),
    ],
    out_specs=[
        pl.BlockSpec((R_BLK, 1), lambda i: (i, 0)),
        pl.BlockSpec((R_BLK, 1), lambda i: (i, 0)),
    ],
    out_shape=[
        jax.ShapeDtypeStruct((N, 1), f32),
        jax.ShapeDtypeStruct((N, 1), f32),
    ],
)


def _final_body(st_hbm, dinv_hbm, q_hbm, src_hbm, dst_hbm, out_hbm,
                s_v, src_v, dst_v, vals_v, zz_v, zbuf, dvbuf, qbuf, obuf, acc_s,
                semf0, semf1, semf2, semf3):
    cid = lax.axis_index("c")
    sid = lax.axis_index("s")

    @pl.when(cid == 0)
    def _core0():
        _zero_1d(zz_v, 63)

        @pl.when(sid < 10)
        def _zero_acc():
            pltpu.sync_copy(zz_v.at[pl.ds(0, 1000)], acc_s.at[pl.ds(sid * 1000, 1000)])

        plsc.subcore_barrier()
        pltpu.sync_copy(st_hbm, s_v)
        pltpu.sync_copy(src_hbm.at[pl.ds(sid * ET, ET)], src_v)
        pltpu.sync_copy(dst_hbm.at[sid], dst_v)

        semf = [semf0, semf1, semf2, semf3]

        def blk(i, _):
            j = 4 * i
            for p in range(4):
                jb = j + p
                for k in range(5):
                    idx = src_v[pl.ds(jb * BA + k * 16, 16)]
                    vals_v[pl.ds(jb * BA + k * 16, 16)] = plsc.load_gather(s_v, [idx])

                @pl.when(i > 0)
                def _w(p=p, jb=jb):
                    pltpu.make_async_copy(vals_v.at[pl.ds(0, BA)],
                                          acc_s.at[dst_v.at[jb]], semf[p]).wait()

                pltpu.async_copy(vals_v.at[pl.ds(jb * BA, BA)],
                                 acc_s.at[dst_v.at[jb]], semf[p], add=True)
            return 0
        lax.fori_loop(0, 62, blk, 0)
        for t in range(2):
            jb = 248 + t
            for k in range(5):
                idx = src_v[pl.ds(jb * BA + k * 16, 16)]
                vals_v[pl.ds(jb * BA + k * 16, 16)] = plsc.load_gather(s_v, [idx])
            pltpu.make_async_copy(vals_v.at[pl.ds(0, BA)],
                                  acc_s.at[dst_v.at[jb]], semf[t]).wait()
            pltpu.async_copy(vals_v.at[pl.ds(jb * BA, BA)],
                             acc_s.at[dst_v.at[jb]], semf[t], add=True)
        for p in range(4):
            pltpu.make_async_copy(vals_v.at[pl.ds(0, BA)],
                                  acc_s.at[dst_v.at[0]], semf[p]).wait()
        plsc.subcore_barrier()

        nbase = sid * 640

        @pl.when(sid < 15)
        def _ld_full():
            pltpu.sync_copy(acc_s.at[pl.ds(nbase, 640)], zbuf)
            pltpu.sync_copy(dinv_hbm.at[pl.ds(nbase, 640)], dvbuf)
            pltpu.sync_copy(q_hbm.at[pl.ds(nbase, 640)], qbuf)

        @pl.when(sid == 15)
        def _ld_tail():
            pltpu.sync_copy(acc_s.at[pl.ds(9600, 400)], zbuf.at[pl.ds(0, 400)])
            pltpu.sync_copy(dinv_hbm.at[pl.ds(9600, 400)], dvbuf.at[pl.ds(0, 400)])
            pltpu.sync_copy(q_hbm.at[pl.ds(9600, 400)], qbuf.at[pl.ds(0, 400)])

        nvec = jnp.where(sid < 15, 40, 25)

        def fin(i, _):
            t = dvbuf[pl.ds(i * 16, 16)] * zbuf[pl.ds(i * 16, 16)] + qbuf[pl.ds(i * 16, 16)]
            obuf[pl.ds(i * 16, 16)] = 1.0 / (1.0 + jnp.exp(-t))
            return 0
        lax.fori_loop(0, nvec, fin, 0)

        @pl.when(sid < 15)
        def _st_full():
            pltpu.sync_copy(obuf, out_hbm.at[pl.ds(nbase, 640)])

        @pl.when(sid == 15)
        def _st_tail():
            pltpu.sync_copy(obuf.at[pl.ds(0, 400)], out_hbm.at[pl.ds(9600, 400)])


_final = pl.kernel(
    _final_body,
    out_type=jax.ShapeDtypeStruct((N,), f32),
    mesh=_MESH,
    compiler_params=pltpu.CompilerParams(needs_layout_passes=False),
    scratch_types=[
        pltpu.VMEM((N,), f32),
        pltpu.VMEM((ET,), i32),
        pltpu.VMEM((NBF, BA), i32),
        pltpu.VMEM((ET,), f32),
        pltpu.VMEM((1008,), f32),
        pltpu.VMEM((640,), f32),
        pltpu.VMEM((640,), f32),
        pltpu.VMEM((640,), f32),
        pltpu.VMEM((640,), f32),
        pltpu.VMEM_SHARED((N,), f32),
        pltpu.SemaphoreType.DMA,
        pltpu.SemaphoreType.DMA,
        pltpu.SemaphoreType.DMA,
        pltpu.SemaphoreType.DMA,
    ],
)


def kernel(x, edge_index, W1, b1, W2, b2, Wlin, blin):
    src = edge_index[0].astype(i32)
    dst = edge_index[1].astype(i32)
    src3 = src.reshape(NC, NS, NB, BA)
    dst3 = dst.reshape(NC, NS, NB, BA)
    src5d = src.reshape(NC, NS, 5, 25, BA)
    dst5d = dst.reshape(NC, NS, 5, 25, BA)
    dst5 = dst.reshape(NS, NBF, BA)

    deg = _deg(dst3)
    d0 = deg[:N].reshape(N, 1)
    d1 = deg[N:].reshape(N, 1)
    xt, dinv = _prescale(x, d0, d1)
    yp = _prop(xt, src5d, dst5d)
    y0 = yp[:N]
    y1 = yp[N:]
    st, q = _dense(y0, y1, xt, dinv, W1, b1.reshape(1, D), W2,
                   Wlin, b2.reshape(1, D), blin.reshape(1, 1))
    out = _final(st.reshape(N), dinv.reshape(N), q.reshape(N), src, dst5)
    return out.reshape(N, 1)


# speedup vs baseline: 43.4440x; 1.1860x over previous
import jax
import jax.numpy as jnp
from jax import lax
from jax.experimental import pallas as pl
from jax.experimental.pallas import tpu as pltpu
from jax.experimental.pallas import tpu_sc as plsc

N = 10000
D = 128
E = 320000
NC = 2
NS = 16
BA = 80
BB = 40
NB = (E // (NC * NS)) // BA
NBF = (E // NS) // BA
ET = E // NS

f32 = jnp.float32
i32 = jnp.int32

_MESH = plsc.VectorSubcoreMesh(core_axis_name="c", subcore_axis_name="s")


def _zero_1d(ref, nvec):
    def z(i, _):
        ref[pl.ds(i * 16, 16)] = jnp.zeros((16,), f32)
        return 0
    lax.fori_loop(0, nvec, z, 0)


def _deg_body(dst_hbm, deg_out, dst_v, ones_v, buf_v, acc_s,
              semd0, semd1, semd2, semd3):
    cid = lax.axis_index("c")
    sid = lax.axis_index("s")
    _zero_1d(buf_v, 63)
    for i in range(BA // 16):
        ones_v[pl.ds(i * 16, 16)] = jnp.ones((16,), f32)

    @pl.when(sid < 10)
    def _zero_acc():
        pltpu.sync_copy(buf_v.at[pl.ds(0, 1000)], acc_s.at[pl.ds(sid * 1000, 1000)])

    plsc.subcore_barrier()
    pltpu.sync_copy(dst_hbm.at[cid, sid], dst_v)

    semd = [semd0, semd1, semd2, semd3]

    def body(i, _):
        j = 4 * i
        for p in range(4):
            @pl.when(i > 0)
            def _w(p=p):
                pltpu.make_async_copy(ones_v, acc_s.at[dst_v.at[j + p]], semd[p]).wait()
            pltpu.async_copy(ones_v, acc_s.at[dst_v.at[j + p]], semd[p], add=True)
        return 0
    lax.fori_loop(0, 31, body, 0)
    pltpu.make_async_copy(ones_v, acc_s.at[dst_v.at[124]], semd0).wait()
    pltpu.async_copy(ones_v, acc_s.at[dst_v.at[124]], semd0, add=True)
    for p in range(4):
        pltpu.make_async_copy(ones_v, acc_s.at[dst_v.at[124]], semd[p]).wait()
    plsc.subcore_barrier()

    @pl.when(sid < 10)
    def _drain():
        pltpu.sync_copy(acc_s.at[pl.ds(sid * 1000, 1000)], buf_v.at[pl.ds(0, 1000)])
        pltpu.sync_copy(buf_v.at[pl.ds(0, 1000)],
                        deg_out.at[pl.ds(cid * N + sid * 1000, 1000)])


_deg = pl.kernel(
    _deg_body,
    out_type=jax.ShapeDtypeStruct((2 * N,), f32),
    mesh=_MESH,
    scratch_types=[
        pltpu.VMEM((NB, BA), i32),
        pltpu.VMEM((BA,), f32),
        pltpu.VMEM((1008,), f32),
        pltpu.VMEM_SHARED((N,), f32),
        pltpu.SemaphoreType.DMA,
        pltpu.SemaphoreType.DMA,
        pltpu.SemaphoreType.DMA,
        pltpu.SemaphoreType.DMA,
    ],
)


R_BLK = 2000


def _prescale_body(x_ref, d0_ref, d1_ref, xt_ref, dinv_ref):
    dv = lax.rsqrt(d0_ref[...] + d1_ref[...] + 1.0)
    dinv_ref[...] = dv
    xt_ref[...] = x_ref[...] * dv


_prescale = pl.pallas_call(
    _prescale_body,
    grid=(N // R_BLK,),
    in_specs=[
        pl.BlockSpec((R_BLK, D), lambda i: (i, 0)),
        pl.BlockSpec((R_BLK, 1), lambda i: (i, 0)),
        pl.BlockSpec((R_BLK, 1), lambda i: (i, 0)),
    ],
    out_specs=[
        pl.BlockSpec((R_BLK, D), lambda i: (i, 0)),
        pl.BlockSpec((R_BLK, 1), lambda i: (i, 0)),
    ],
    out_shape=[
        jax.ShapeDtypeStruct((N, D), f32),
        jax.ShapeDtypeStruct((N, 1), f32),
    ],
)


def _prop_body(xt_hbm, src_hbm, dst_hbm, y_out,
               src_c, dst_c, r0, r1, r2, r3, acc_s,
               g0, g1, g2, g3, s0, s1, s2, s3):
    cid = lax.axis_index("c")
    sid = lax.axis_index("s")
    rows = [r0, r1, r2, r3]
    gsem = [g0, g1, g2, g3]
    ssem = [s0, s1, s2, s3]

    def zb(i, _):
        r0[i // 8, pl.ds((i % 8) * 16, 16)] = jnp.zeros((16,), f32)
        return 0
    lax.fori_loop(0, BB * 8, zb, 0)

    @pl.when(sid < 10)
    def _zacc():
        def zacc(k, _):
            pltpu.async_copy(r0, acc_s.at[pl.ds(sid * 1000 + k * BB, BB)], g0)
            return 0
        lax.fori_loop(0, 25, zacc, 0)

        def zw(k, _):
            pltpu.make_async_copy(r0, acc_s.at[pl.ds(sid * 1000, BB)], g0).wait()
            return 0
        lax.fori_loop(0, 25, zw, 0)

    plsc.subcore_barrier()

    def chunk(m, _):
        pltpu.sync_copy(src_hbm.at[cid, sid, m], src_c)
        pltpu.sync_copy(dst_hbm.at[cid, sid, m], dst_c)
        for p in range(3):
            pltpu.async_copy(xt_hbm.at[src_c.at[p]], rows[p], gsem[p])

        def step(i, _):
            for p in range(4):
                jj = 4 * i + p
                q = (p + 3) % 4
                pltpu.make_async_copy(xt_hbm.at[src_c.at[jj]], rows[p], gsem[p]).wait()
                if p == 0:
                    @pl.when(i > 0)
                    def _ws():
                        pltpu.make_async_copy(rows[q], acc_s.at[dst_c.at[jj]], ssem[q]).wait()
                else:
                    pltpu.make_async_copy(rows[q], acc_s.at[dst_c.at[jj]], ssem[q]).wait()
                if p < 2:
                    pltpu.async_copy(xt_hbm.at[src_c.at[jj + 3]], rows[q], gsem[q])
                else:
                    @pl.when(i < 5)
                    def _ig():
                        pltpu.async_copy(xt_hbm.at[src_c.at[jj + 3]], rows[q], gsem[q])
                pltpu.async_copy(rows[p], acc_s.at[dst_c.at[jj]], ssem[p], add=True)
            return 0
        lax.fori_loop(0, 6, step, 0)
        pltpu.make_async_copy(xt_hbm.at[src_c.at[24]], rows[0], gsem[0]).wait()
        pltpu.make_async_copy(rows[3], acc_s.at[dst_c.at[24]], ssem[3]).wait()
        pltpu.async_copy(rows[0], acc_s.at[dst_c.at[24]], ssem[0], add=True).wait()
        return 0
    lax.fori_loop(0, 10, chunk, 0)
    plsc.subcore_barrier()

    @pl.when(sid < 10)
    def _drain():
        pltpu.async_copy(acc_s.at[pl.ds(sid * 1000, BB)], r0, g0)

        def drain(k, _):
            base = sid * 1000 + k * BB

            def dk(rbuf, gs):
                pltpu.make_async_copy(acc_s.at[pl.ds(base, BB)], rbuf, gs).wait()

                @pl.when(k < 24)
                def _nx():
                    pltpu.async_copy(
                        acc_s.at[pl.ds(base + BB, BB)],
                        r1 if rbuf is r0 else r0,
                        g1 if rbuf is r0 else g0)
                pltpu.sync_copy(rbuf, y_out.at[pl.ds(cid * N + base, BB)])

            @pl.when(k % 2 == 0)
            def _e():
                dk(r0, g0)

            @pl.when(k % 2 == 1)
            def _o():
                dk(r1, g1)
            return 0
        lax.fori_loop(0, 25, drain, 0)


_prop = pl.kernel(
    _prop_body,
    out_type=jax.ShapeDtypeStruct((2 * N, D), f32),
    mesh=_MESH,
    compiler_params=pltpu.CompilerParams(needs_layout_passes=False),
    scratch_types=[
        pltpu.VMEM((25, BB), i32),
        pltpu.VMEM((25, BB), i32),
        pltpu.VMEM((BB, D), f32),
        pltpu.VMEM((BB, D), f32),
        pltpu.VMEM((BB, D), f32),
        pltpu.VMEM((BB, D), f32),
        pltpu.VMEM_SHARED((N, D), f32),
        pltpu.SemaphoreType.DMA,
        pltpu.SemaphoreType.DMA,
        pltpu.SemaphoreType.DMA,
        pltpu.SemaphoreType.DMA,
        pltpu.SemaphoreType.DMA,
        pltpu.SemaphoreType.DMA,
        pltpu.SemaphoreType.DMA,
        pltpu.SemaphoreType.DMA,
    ],
)


def _dense_body(y0_ref, y1_ref, xt_ref, dv_ref, W1_ref, b1_ref, W2_ref,
                Wl_ref, b2_ref, bl_ref, s_ref, q_ref):
    dv = dv_ref[...]
    y = dv * (y0_ref[...] + y1_ref[...] + xt_ref[...])
    h = jnp.maximum(
        jnp.dot(y, W1_ref[...], preferred_element_type=f32) + b1_ref[...], 0.0)
    w = jnp.dot(W2_ref[...], Wl_ref[...], preferred_element_type=f32)
    st = dv * jnp.dot(h, w, preferred_element_type=f32)
    s_ref[...] = st
    c = jnp.dot(b2_ref[...], Wl_ref[...], preferred_element_type=f32) + bl_ref[...]
    q_ref[...] = dv * st + c


_dense = pl.pallas_call(
    _dense_body,
    grid=(N // R_BLK,),
    in_specs=[
        pl.BlockSpec((R_BLK, D), lambda i: (i, 0)),
        pl.BlockSpec((R_BLK, D), lambda i: (i, 0)),
        pl.BlockSpec((R_BLK, D), lambda i: (i, 0)),
        pl.BlockSpec((R_BLK, 1), lambda i: (i, 0)),
        pl.BlockSpec((D, D), lambda i: (0, 0)),
        pl.BlockSpec((1, D), lambda i: (0, 0)),
        pl.BlockSpec((D, D), lambda i: (0, 0)),
        pl.BlockSpec((D, 1), lambda i: (0, 0)),
        pl.BlockSpec((1, D), lambda i: (0, 0)),
        pl.BlockSpec((1, 1), lambda i: (0, 0)),
    ],
    out_specs=[
        pl.BlockSpec((R_BLK, 1), lambda i: (i, 0)),
        pl.BlockSpec((R_BLK, 1), lambda i: (i, 0)),
    ],
    out_shape=[
        jax.ShapeDtypeStruct((N, 1), f32),
        jax.ShapeDtypeStruct((N, 1), f32),
    ],
)


def _final_body(st_hbm, dinv_hbm, q_hbm, src_hbm, dst_hbm, out_hbm,
                s_v, src_v, dst_v, vals_v, zz_v, zbuf, dvbuf, qbuf, obuf, acc_s,
                semf0, semf1, semf2, semf3):
    cid = lax.axis_index("c")
    sid = lax.axis_index("s")

    @pl.when(cid == 0)
    def _core0():
        _zero_1d(zz_v, 63)

        @pl.when(sid < 10)
        def _zero_acc():
            pltpu.sync_copy(zz_v.at[pl.ds(0, 1000)], acc_s.at[pl.ds(sid * 1000, 1000)])

        plsc.subcore_barrier()
        pltpu.sync_copy(st_hbm, s_v)
        pltpu.sync_copy(src_hbm.at[pl.ds(sid * ET, ET)], src_v)
        pltpu.sync_copy(dst_hbm.at[sid], dst_v)

        semf = [semf0, semf1, semf2, semf3]

        def blk(i, _):
            j = 4 * i
            for p in range(4):
                jb = j + p
                for k in range(5):
                    idx = src_v[pl.ds(jb * BA + k * 16, 16)]
                    vals_v[pl.ds(jb * BA + k * 16, 16)] = plsc.load_gather(s_v, [idx])

                @pl.when(i > 0)
                def _w(p=p, jb=jb):
                    pltpu.make_async_copy(vals_v.at[pl.ds(0, BA)],
                                          acc_s.at[dst_v.at[jb]], semf[p]).wait()

                pltpu.async_copy(vals_v.at[pl.ds(jb * BA, BA)],
                                 acc_s.at[dst_v.at[jb]], semf[p], add=True)
            return 0
        lax.fori_loop(0, 62, blk, 0)
        for t in range(2):
            jb = 248 + t
            for k in range(5):
                idx = src_v[pl.ds(jb * BA + k * 16, 16)]
                vals_v[pl.ds(jb * BA + k * 16, 16)] = plsc.load_gather(s_v, [idx])
            pltpu.make_async_copy(vals_v.at[pl.ds(0, BA)],
                                  acc_s.at[dst_v.at[jb]], semf[t]).wait()
            pltpu.async_copy(vals_v.at[pl.ds(jb * BA, BA)],
                             acc_s.at[dst_v.at[jb]], semf[t], add=True)
        for p in range(4):
            pltpu.make_async_copy(vals_v.at[pl.ds(0, BA)],
                                  acc_s.at[dst_v.at[0]], semf[p]).wait()
        plsc.subcore_barrier()

        nbase = sid * 640

        @pl.when(sid < 15)
        def _ld_full():
            pltpu.sync_copy(acc_s.at[pl.ds(nbase, 640)], zbuf)
            pltpu.sync_copy(dinv_hbm.at[pl.ds(nbase, 640)], dvbuf)
            pltpu.sync_copy(q_hbm.at[pl.ds(nbase, 640)], qbuf)

        @pl.when(sid == 15)
        def _ld_tail():
            pltpu.sync_copy(acc_s.at[pl.ds(9600, 400)], zbuf.at[pl.ds(0, 400)])
            pltpu.sync_copy(dinv_hbm.at[pl.ds(9600, 400)], dvbuf.at[pl.ds(0, 400)])
            pltpu.sync_copy(q_hbm.at[pl.ds(9600, 400)], qbuf.at[pl.ds(0, 400)])

        nvec = jnp.where(sid < 15, 40, 25)

        def fin(i, _):
            t = dvbuf[pl.ds(i * 16, 16)] * zbuf[pl.ds(i * 16, 16)] + qbuf[pl.ds(i * 16, 16)]
            obuf[pl.ds(i * 16, 16)] = 1.0 / (1.0 + jnp.exp(-t))
            return 0
        lax.fori_loop(0, nvec, fin, 0)

        @pl.when(sid < 15)
        def _st_full():
            pltpu.sync_copy(obuf, out_hbm.at[pl.ds(nbase, 640)])

        @pl.when(sid == 15)
        def _st_tail():
            pltpu.sync_copy(obuf.at[pl.ds(0, 400)], out_hbm.at[pl.ds(9600, 400)])


_final = pl.kernel(
    _final_body,
    out_type=jax.ShapeDtypeStruct((N,), f32),
    mesh=_MESH,
    compiler_params=pltpu.CompilerParams(needs_layout_passes=False),
    scratch_types=[
        pltpu.VMEM((N,), f32),
        pltpu.VMEM((ET,), i32),
        pltpu.VMEM((NBF, BA), i32),
        pltpu.VMEM((ET,), f32),
        pltpu.VMEM((1008,), f32),
        pltpu.VMEM((640,), f32),
        pltpu.VMEM((640,), f32),
        pltpu.VMEM((640,), f32),
        pltpu.VMEM((640,), f32),
        pltpu.VMEM_SHARED((N,), f32),
        pltpu.SemaphoreType.DMA,
        pltpu.SemaphoreType.DMA,
        pltpu.SemaphoreType.DMA,
        pltpu.SemaphoreType.DMA,
    ],
)


def kernel(x, edge_index, W1, b1, W2, b2, Wlin, blin):
    src = edge_index[0].astype(i32)
    dst = edge_index[1].astype(i32)
    src3 = src.reshape(NC, NS, NB, BA)
    dst3 = dst.reshape(NC, NS, NB, BA)
    src5d = src.reshape(NC, NS, 10, 25, BB)
    dst5d = dst.reshape(NC, NS, 10, 25, BB)
    dst5 = dst.reshape(NS, NBF, BA)

    deg = _deg(dst3)
    d0 = deg[:N].reshape(N, 1)
    d1 = deg[N:].reshape(N, 1)
    xt, dinv = _prescale(x, d0, d1)
    yp = _prop(xt, src5d, dst5d)
    y0 = yp[:N]
    y1 = yp[N:]
    st, q = _dense(y0, y1, xt, dinv, W1, b1.reshape(1, D), W2,
                   Wlin, b2.reshape(1, D), blin.reshape(1, 1))
    out = _final(st.reshape(N), dinv.reshape(N), q.reshape(N), src, dst5)
    return out.reshape(N, 1)


# speedup vs baseline: 44.7309x; 1.0296x over previous
import jax
import jax.numpy as jnp
from jax import lax
from jax.experimental import pallas as pl
from jax.experimental.pallas import tpu as pltpu
from jax.experimental.pallas import tpu_sc as plsc

N = 10000
D = 128
E = 320000
NC = 2
NS = 16
BA = 80
BB = 40
NB = (E // (NC * NS)) // BA
NBF = (E // NS) // BA
ET = E // NS

f32 = jnp.float32
i32 = jnp.int32

_MESH = plsc.VectorSubcoreMesh(core_axis_name="c", subcore_axis_name="s")


def _zero_1d(ref, nvec):
    def z(i, _):
        ref[pl.ds(i * 16, 16)] = jnp.zeros((16,), f32)
        return 0
    lax.fori_loop(0, nvec, z, 0)


def _deg_body(dst_hbm, deg_out, dst_v, ones_v, buf_v, acc_s,
              semd0, semd1, semd2, semd3):
    cid = lax.axis_index("c")
    sid = lax.axis_index("s")
    _zero_1d(buf_v, 63)
    for i in range(BA // 16):
        ones_v[pl.ds(i * 16, 16)] = jnp.ones((16,), f32)

    @pl.when(sid < 10)
    def _zero_acc():
        pltpu.sync_copy(buf_v.at[pl.ds(0, 1000)], acc_s.at[pl.ds(sid * 1000, 1000)])

    plsc.subcore_barrier()
    pltpu.sync_copy(dst_hbm.at[cid, sid], dst_v)

    semd = [semd0, semd1, semd2, semd3]

    def body(i, _):
        j = 4 * i
        for p in range(4):
            @pl.when(i > 0)
            def _w(p=p):
                pltpu.make_async_copy(ones_v, acc_s.at[dst_v.at[j + p]], semd[p]).wait()
            pltpu.async_copy(ones_v, acc_s.at[dst_v.at[j + p]], semd[p], add=True)
        return 0
    lax.fori_loop(0, 31, body, 0)
    pltpu.make_async_copy(ones_v, acc_s.at[dst_v.at[124]], semd0).wait()
    pltpu.async_copy(ones_v, acc_s.at[dst_v.at[124]], semd0, add=True)
    for p in range(4):
        pltpu.make_async_copy(ones_v, acc_s.at[dst_v.at[124]], semd[p]).wait()
    plsc.subcore_barrier()

    @pl.when(sid < 10)
    def _drain():
        pltpu.sync_copy(acc_s.at[pl.ds(sid * 1000, 1000)], buf_v.at[pl.ds(0, 1000)])
        pltpu.sync_copy(buf_v.at[pl.ds(0, 1000)],
                        deg_out.at[pl.ds(cid * N + sid * 1000, 1000)])


_deg = pl.kernel(
    _deg_body,
    out_type=jax.ShapeDtypeStruct((2 * N,), f32),
    mesh=_MESH,
    scratch_types=[
        pltpu.VMEM((NB, BA), i32),
        pltpu.VMEM((BA,), f32),
        pltpu.VMEM((1008,), f32),
        pltpu.VMEM_SHARED((N,), f32),
        pltpu.SemaphoreType.DMA,
        pltpu.SemaphoreType.DMA,
        pltpu.SemaphoreType.DMA,
        pltpu.SemaphoreType.DMA,
    ],
)


R_BLK = 2000


def _prescale_body(x_ref, d0_ref, d1_ref, xt_ref, dinv_ref):
    dv = lax.rsqrt(d0_ref[...] + d1_ref[...] + 1.0)
    dinv_ref[...] = dv
    xt_ref[...] = x_ref[...] * dv


_prescale = pl.pallas_call(
    _prescale_body,
    grid=(N // R_BLK,),
    in_specs=[
        pl.BlockSpec((R_BLK, D), lambda i: (i, 0)),
        pl.BlockSpec((R_BLK, 1), lambda i: (i, 0)),
        pl.BlockSpec((R_BLK, 1), lambda i: (i, 0)),
    ],
    out_specs=[
        pl.BlockSpec((R_BLK, D), lambda i: (i, 0)),
        pl.BlockSpec((R_BLK, 1), lambda i: (i, 0)),
    ],
    out_shape=[
        jax.ShapeDtypeStruct((N, D), f32),
        jax.ShapeDtypeStruct((N, 1), f32),
    ],
)


def _prop_body(xt_hbm, src_hbm, dst_hbm, y_out,
               src_c, dst_c, r0, r1, r2, r3, r4, acc_s,
               g0, g1, g2, g3, g4, s0, s1, s2, s3, s4):
    cid = lax.axis_index("c")
    sid = lax.axis_index("s")
    rows = [r0, r1, r2, r3, r4]
    gsem = [g0, g1, g2, g3, g4]
    ssem = [s0, s1, s2, s3, s4]

    def zb(i, _):
        r0[i // 8, pl.ds((i % 8) * 16, 16)] = jnp.zeros((16,), f32)
        return 0
    lax.fori_loop(0, BB * 8, zb, 0)

    @pl.when(sid < 10)
    def _zacc():
        def zacc(k, _):
            pltpu.async_copy(r0, acc_s.at[pl.ds(sid * 1000 + k * BB, BB)], g0)
            return 0
        lax.fori_loop(0, 25, zacc, 0)

        def zw(k, _):
            pltpu.make_async_copy(r0, acc_s.at[pl.ds(sid * 1000, BB)], g0).wait()
            return 0
        lax.fori_loop(0, 25, zw, 0)

    plsc.subcore_barrier()

    def chunk(m, _):
        pltpu.sync_copy(src_hbm.at[cid, sid, m], src_c)
        pltpu.sync_copy(dst_hbm.at[cid, sid, m], dst_c)
        for p in range(4):
            pltpu.async_copy(xt_hbm.at[src_c.at[p]], rows[p], gsem[p])

        def step(i, _):
            for p in range(5):
                jj = 5 * i + p
                q = (p + 4) % 5
                pltpu.make_async_copy(xt_hbm.at[src_c.at[jj]], rows[p], gsem[p]).wait()
                if p == 0:
                    @pl.when(i > 0)
                    def _ws():
                        pltpu.make_async_copy(rows[q], acc_s.at[dst_c.at[jj]], ssem[q]).wait()
                else:
                    pltpu.make_async_copy(rows[q], acc_s.at[dst_c.at[jj]], ssem[q]).wait()
                if p == 0:
                    pltpu.async_copy(xt_hbm.at[src_c.at[jj + 4]], rows[q], gsem[q])
                else:
                    @pl.when(i < 4)
                    def _ig():
                        pltpu.async_copy(xt_hbm.at[src_c.at[jj + 4]], rows[q], gsem[q])
                pltpu.async_copy(rows[p], acc_s.at[dst_c.at[jj]], ssem[p], add=True)
            return 0
        lax.fori_loop(0, 5, step, 0)
        pltpu.make_async_copy(rows[4], acc_s.at[dst_c.at[24]], ssem[4]).wait()
        return 0
    lax.fori_loop(0, 10, chunk, 0)
    plsc.subcore_barrier()

    @pl.when(sid < 10)
    def _drain():
        pltpu.async_copy(acc_s.at[pl.ds(sid * 1000, BB)], r0, g0)

        def drain(k, _):
            base = sid * 1000 + k * BB

            def dk(rbuf, gs):
                pltpu.make_async_copy(acc_s.at[pl.ds(base, BB)], rbuf, gs).wait()

                @pl.when(k < 24)
                def _nx():
                    pltpu.async_copy(
                        acc_s.at[pl.ds(base + BB, BB)],
                        r1 if rbuf is r0 else r0,
                        g1 if rbuf is r0 else g0)
                pltpu.sync_copy(rbuf, y_out.at[pl.ds(cid * N + base, BB)])

            @pl.when(k % 2 == 0)
            def _e():
                dk(r0, g0)

            @pl.when(k % 2 == 1)
            def _o():
                dk(r1, g1)
            return 0
        lax.fori_loop(0, 25, drain, 0)


_prop = pl.kernel(
    _prop_body,
    out_type=jax.ShapeDtypeStruct((2 * N, D), f32),
    mesh=_MESH,
    compiler_params=pltpu.CompilerParams(needs_layout_passes=False),
    scratch_types=[
        pltpu.VMEM((25, BB), i32),
        pltpu.VMEM((25, BB), i32),
        pltpu.VMEM((BB, D), f32),
        pltpu.VMEM((BB, D), f32),
        pltpu.VMEM((BB, D), f32),
        pltpu.VMEM((BB, D), f32),
        pltpu.VMEM((BB, D), f32),
        pltpu.VMEM_SHARED((N, D), f32),
        pltpu.SemaphoreType.DMA,
        pltpu.SemaphoreType.DMA,
        pltpu.SemaphoreType.DMA,
        pltpu.SemaphoreType.DMA,
        pltpu.SemaphoreType.DMA,
        pltpu.SemaphoreType.DMA,
        pltpu.SemaphoreType.DMA,
        pltpu.SemaphoreType.DMA,
        pltpu.SemaphoreType.DMA,
        pltpu.SemaphoreType.DMA,
    ],
)


def _dense_body(y0_ref, y1_ref, xt_ref, dv_ref, W1_ref, b1_ref, W2_ref,
                Wl_ref, b2_ref, bl_ref, s_ref, q_ref):
    dv = dv_ref[...]
    y = dv * (y0_ref[...] + y1_ref[...] + xt_ref[...])
    h = jnp.maximum(
        jnp.dot(y, W1_ref[...], preferred_element_type=f32) + b1_ref[...], 0.0)
    w = jnp.dot(W2_ref[...], Wl_ref[...], preferred_element_type=f32)
    st = dv * jnp.dot(h, w, preferred_element_type=f32)
    s_ref[...] = st
    c = jnp.dot(b2_ref[...], Wl_ref[...], preferred_element_type=f32) + bl_ref[...]
    q_ref[...] = dv * st + c


_dense = pl.pallas_call(
    _dense_body,
    grid=(N // R_BLK,),
    in_specs=[
        pl.BlockSpec((R_BLK, D), lambda i: (i, 0)),
        pl.BlockSpec((R_BLK, D), lambda i: (i, 0)),
        pl.BlockSpec((R_BLK, D), lambda i: (i, 0)),
        pl.BlockSpec((R_BLK, 1), lambda i: (i, 0)),
        pl.BlockSpec((D, D), lambda i: (0, 0)),
        pl.BlockSpec((1, D), lambda i: (0, 0)),
        pl.BlockSpec((D, D), lambda i: (0, 0)),
        pl.BlockSpec((D, 1), lambda i: (0, 0)),
        pl.BlockSpec((1, D), lambda i: (0, 0)),
        pl.BlockSpec((1, 1), lambda i: (0, 0)),
    ],
    out_specs=[
        pl.BlockSpec((R_BLK, 1), lambda i: (i, 0)),
        pl.BlockSpec((R_BLK, 1), lambda i: (i, 0)),
    ],
    out_shape=[
        jax.ShapeDtypeStruct((N, 1), f32),
        jax.ShapeDtypeStruct((N, 1), f32),
    ],
)


def _final_body(st_hbm, dinv_hbm, q_hbm, src_hbm, dst_hbm, out_hbm,
                s_v, src_v, dst_v, vals_v, zz_v, zbuf, dvbuf, qbuf, obuf, acc_s,
                semf0, semf1, semf2, semf3):
    cid = lax.axis_index("c")
    sid = lax.axis_index("s")

    @pl.when(cid == 0)
    def _core0():
        _zero_1d(zz_v, 63)

        @pl.when(sid < 10)
        def _zero_acc():
            pltpu.sync_copy(zz_v.at[pl.ds(0, 1000)], acc_s.at[pl.ds(sid * 1000, 1000)])

        plsc.subcore_barrier()
        pltpu.sync_copy(st_hbm, s_v)
        pltpu.sync_copy(src_hbm.at[pl.ds(sid * ET, ET)], src_v)
        pltpu.sync_copy(dst_hbm.at[sid], dst_v)

        semf = [semf0, semf1, semf2, semf3]

        def blk(i, _):
            j = 4 * i
            for p in range(4):
                jb = j + p
                for k in range(5):
                    idx = src_v[pl.ds(jb * BA + k * 16, 16)]
                    vals_v[pl.ds(jb * BA + k * 16, 16)] = plsc.load_gather(s_v, [idx])

                @pl.when(i > 0)
                def _w(p=p, jb=jb):
                    pltpu.make_async_copy(vals_v.at[pl.ds(0, BA)],
                                          acc_s.at[dst_v.at[jb]], semf[p]).wait()

                pltpu.async_copy(vals_v.at[pl.ds(jb * BA, BA)],
                                 acc_s.at[dst_v.at[jb]], semf[p], add=True)
            return 0
        lax.fori_loop(0, 62, blk, 0)
        for t in range(2):
            jb = 248 + t
            for k in range(5):
                idx = src_v[pl.ds(jb * BA + k * 16, 16)]
                vals_v[pl.ds(jb * BA + k * 16, 16)] = plsc.load_gather(s_v, [idx])
            pltpu.make_async_copy(vals_v.at[pl.ds(0, BA)],
                                  acc_s.at[dst_v.at[jb]], semf[t]).wait()
            pltpu.async_copy(vals_v.at[pl.ds(jb * BA, BA)],
                             acc_s.at[dst_v.at[jb]], semf[t], add=True)
        for p in range(4):
            pltpu.make_async_copy(vals_v.at[pl.ds(0, BA)],
                                  acc_s.at[dst_v.at[0]], semf[p]).wait()
        plsc.subcore_barrier()

        nbase = sid * 640

        @pl.when(sid < 15)
        def _ld_full():
            pltpu.sync_copy(acc_s.at[pl.ds(nbase, 640)], zbuf)
            pltpu.sync_copy(dinv_hbm.at[pl.ds(nbase, 640)], dvbuf)
            pltpu.sync_copy(q_hbm.at[pl.ds(nbase, 640)], qbuf)

        @pl.when(sid == 15)
        def _ld_tail():
            pltpu.sync_copy(acc_s.at[pl.ds(9600, 400)], zbuf.at[pl.ds(0, 400)])
            pltpu.sync_copy(dinv_hbm.at[pl.ds(9600, 400)], dvbuf.at[pl.ds(0, 400)])
            pltpu.sync_copy(q_hbm.at[pl.ds(9600, 400)], qbuf.at[pl.ds(0, 400)])

        nvec = jnp.where(sid < 15, 40, 25)

        def fin(i, _):
            t = dvbuf[pl.ds(i * 16, 16)] * zbuf[pl.ds(i * 16, 16)] + qbuf[pl.ds(i * 16, 16)]
            obuf[pl.ds(i * 16, 16)] = 1.0 / (1.0 + jnp.exp(-t))
            return 0
        lax.fori_loop(0, nvec, fin, 0)

        @pl.when(sid < 15)
        def _st_full():
            pltpu.sync_copy(obuf, out_hbm.at[pl.ds(nbase, 640)])

        @pl.when(sid == 15)
        def _st_tail():
            pltpu.sync_copy(obuf.at[pl.ds(0, 400)], out_hbm.at[pl.ds(9600, 400)])


_final = pl.kernel(
    _final_body,
    out_type=jax.ShapeDtypeStruct((N,), f32),
    mesh=_MESH,
    compiler_params=pltpu.CompilerParams(needs_layout_passes=False),
    scratch_types=[
        pltpu.VMEM((N,), f32),
        pltpu.VMEM((ET,), i32),
        pltpu.VMEM((NBF, BA), i32),
        pltpu.VMEM((ET,), f32),
        pltpu.VMEM((1008,), f32),
        pltpu.VMEM((640,), f32),
        pltpu.VMEM((640,), f32),
        pltpu.VMEM((640,), f32),
        pltpu.VMEM((640,), f32),
        pltpu.VMEM_SHARED((N,), f32),
        pltpu.SemaphoreType.DMA,
        pltpu.SemaphoreType.DMA,
        pltpu.SemaphoreType.DMA,
        pltpu.SemaphoreType.DMA,
    ],
)


def kernel(x, edge_index, W1, b1, W2, b2, Wlin, blin):
    src = edge_index[0].astype(i32)
    dst = edge_index[1].astype(i32)
    src3 = src.reshape(NC, NS, NB, BA)
    dst3 = dst.reshape(NC, NS, NB, BA)
    src5d = src.reshape(NC, NS, 10, 25, BB)
    dst5d = dst.reshape(NC, NS, 10, 25, BB)
    dst5 = dst.reshape(NS, NBF, BA)

    deg = _deg(dst3)
    d0 = deg[:N].reshape(N, 1)
    d1 = deg[N:].reshape(N, 1)
    xt, dinv = _prescale(x, d0, d1)
    yp = _prop(xt, src5d, dst5d)
    y0 = yp[:N]
    y1 = yp[N:]
    st, q = _dense(y0, y1, xt, dinv, W1, b1.reshape(1, D), W2,
                   Wlin, b2.reshape(1, D), blin.reshape(1, 1))
    out = _final(st.reshape(N), dinv.reshape(N), q.reshape(N), src, dst5)
    return out.reshape(N, 1)
